# baseline (speedup 1.0000x reference)
import jax
import jax.numpy as jnp
from jax.experimental import pallas as pl
from jax.experimental.pallas import tpu as pltpu

_H = 32


def _body(scalar_ref, hist_ref, act_ref, out_ref, sem):
    shift = scalar_ref[0]
    pos = scalar_ref[1]
    copies = []
    for j in range(_H):
        src = jax.lax.rem(j + shift, _H)
        copies.append(
            pltpu.make_async_copy(
                hist_ref.at[:, pl.ds(src, 1), :],
                out_ref.at[:, pl.ds(j, 1), :],
                sem,
            )
        )
    for c in copies:
        c.start()
    for c in copies:
        c.wait()
    act_copy = pltpu.make_async_copy(act_ref, out_ref.at[:, pl.ds(pos, 1), :], sem)
    act_copy.start()
    act_copy.wait()


def kernel(history, activations, current_idx):
    B, N, H = history.shape
    idx = jnp.asarray(current_idx, dtype=jnp.int32)
    new_idx = idx + 1
    s = new_idx % H
    wrapped = new_idx >= H
    shift = jnp.where(wrapped, s, 0).astype(jnp.int32)
    pos = jnp.where(wrapped, H - 1, idx % H).astype(jnp.int32)
    scalars = jnp.stack([shift, pos])

    hist_t = jnp.transpose(history, (0, 2, 1))
    act3 = activations.reshape(B, 1, N)

    out_t = pl.pallas_call(
        _body,
        grid_spec=pltpu.PrefetchScalarGridSpec(
            num_scalar_prefetch=1,
            grid=(),
            in_specs=[
                pl.BlockSpec(memory_space=pltpu.MemorySpace.HBM),
                pl.BlockSpec(memory_space=pltpu.MemorySpace.HBM),
            ],
            out_specs=pl.BlockSpec(memory_space=pltpu.MemorySpace.HBM),
            scratch_shapes=[pltpu.SemaphoreType.DMA],
        ),
        out_shape=jax.ShapeDtypeStruct((B, H, N), history.dtype),
    )(scalars, hist_t, act3)
    return jnp.transpose(out_t, (0, 2, 1))


# speedup vs baseline: 7.7887x; 7.7887x over previous
import jax
import jax.numpy as jnp
from jax.experimental import pallas as pl
from jax.experimental.pallas import tpu as pltpu

_H = 32
_NB = 65536


def _body(scalar_ref, hist_ref, act_ref, out_ref):
    pos = scalar_ref[1]
    j = pl.program_id(2)

    @pl.when(j != pos)
    def _():
        out_ref[...] = hist_ref[...]

    @pl.when(j == pos)
    def _():
        out_ref[...] = act_ref[...]


def kernel(history, activations, current_idx):
    B, N, H = history.shape
    idx = jnp.asarray(current_idx, dtype=jnp.int32)
    new_idx = idx + 1
    s = new_idx % H
    wrapped = new_idx >= H
    shift = jnp.where(wrapped, s, 0).astype(jnp.int32)
    pos = jnp.where(wrapped, H - 1, idx % H).astype(jnp.int32)
    scalars = jnp.stack([shift, pos])

    h4 = jnp.transpose(history, (0, 2, 1)).reshape(B, H, 1, N)
    a4 = activations.reshape(B, 1, 1, N)

    nb = _NB
    grid = (B, N // nb, H)
    out4 = pl.pallas_call(
        _body,
        grid_spec=pltpu.PrefetchScalarGridSpec(
            num_scalar_prefetch=1,
            grid=grid,
            in_specs=[
                pl.BlockSpec(
                    (1, 1, 1, nb),
                    lambda b, n, j, sref: (b, (j + sref[0]) % _H, 0, n),
                ),
                pl.BlockSpec((1, 1, 1, nb), lambda b, n, j, sref: (b, 0, 0, n)),
            ],
            out_specs=pl.BlockSpec(
                (1, 1, 1, nb), lambda b, n, j, sref: (b, j, 0, n)
            ),
        ),
        out_shape=jax.ShapeDtypeStruct((B, H, 1, N), history.dtype),
    )(scalars, h4, a4)
    return jnp.transpose(out4.reshape(B, H, N), (0, 2, 1))


# speedup vs baseline: 13.4548x; 1.7275x over previous
import jax
import jax.numpy as jnp
from jax.experimental import pallas as pl
from jax.experimental.pallas import tpu as pltpu

_H = 32
_NB = 65536


def _body(scalar_ref, hist_ref, act_ref, out_ref):
    pos = scalar_ref[1]
    j = pl.program_id(0)

    @pl.when(j != pos)
    def _():
        out_ref[...] = hist_ref[...]

    @pl.when(j == pos)
    def _():
        out_ref[...] = act_ref[...]


def kernel(history, activations, current_idx):
    B, N, H = history.shape
    idx = jnp.asarray(current_idx, dtype=jnp.int32)
    new_idx = idx + 1
    s = new_idx % H
    wrapped = new_idx >= H
    shift = jnp.where(wrapped, s, 0).astype(jnp.int32)
    pos = jnp.where(wrapped, H - 1, idx % H).astype(jnp.int32)
    scalars = jnp.stack([shift, pos])

    h4 = jnp.transpose(history, (0, 2, 1)).reshape(B, H, 1, N)
    a4 = activations.reshape(B, 1, 1, N)

    grid = (H,)
    out4 = pl.pallas_call(
        _body,
        grid_spec=pltpu.PrefetchScalarGridSpec(
            num_scalar_prefetch=1,
            grid=grid,
            in_specs=[
                pl.BlockSpec(
                    (B, 1, 1, N),
                    lambda j, sref: (0, (j + sref[0]) % _H, 0, 0),
                ),
                pl.BlockSpec((B, 1, 1, N), lambda j, sref: (0, 0, 0, 0)),
            ],
            out_specs=pl.BlockSpec((B, 1, 1, N), lambda j, sref: (0, j, 0, 0)),
        ),
        out_shape=jax.ShapeDtypeStruct((B, H, 1, N), history.dtype),
    )(scalars, h4, a4)
    return jnp.transpose(out4.reshape(B, H, N), (0, 2, 1))


# speedup vs baseline: 49.6068x; 3.6869x over previous
import jax
import jax.numpy as jnp
from jax.experimental import pallas as pl
from jax.experimental.pallas import tpu as pltpu

_H = 32
_D = 4


def _body(scalar_ref, hist_ref, act_ref, out_ref, buf_in, buf_out, in_sem, out_sem):
    shift = scalar_ref[0]
    pos = scalar_ref[1]

    def start_in(j, b):
        src = jax.lax.rem(j + shift, _H)

        @pl.when(j != pos)
        def _():
            pltpu.make_async_copy(
                hist_ref.at[:, src, :], buf_in.at[b], in_sem.at[b]
            ).start()

        @pl.when(j == pos)
        def _():
            pltpu.make_async_copy(act_ref, buf_in.at[b], in_sem.at[b]).start()

    def wait_in(b):
        pltpu.make_async_copy(act_ref, buf_in.at[b], in_sem.at[b]).wait()

    def start_out(j, b):
        pltpu.make_async_copy(
            buf_out.at[b], out_ref.at[:, j, :], out_sem.at[b]
        ).start()

    def wait_out(j, b):
        pltpu.make_async_copy(
            buf_out.at[b], out_ref.at[:, j, :], out_sem.at[b]
        ).wait()

    for j in range(_D):
        start_in(j, j)
    for j in range(_H):
        b = j % _D
        wait_in(b)
        if j >= _D:
            wait_out(j - _D, b)
        buf_out[b] = buf_in[b]
        start_out(j, b)
        if j + _D < _H:
            start_in(j + _D, b)
    for j in range(_H - _D, _H):
        wait_out(j, j % _D)


def kernel(history, activations, current_idx):
    B, N, H = history.shape
    idx = jnp.asarray(current_idx, dtype=jnp.int32)
    new_idx = idx + 1
    s = new_idx % H
    wrapped = new_idx >= H
    shift = jnp.where(wrapped, s, 0).astype(jnp.int32)
    pos = jnp.where(wrapped, H - 1, idx % H).astype(jnp.int32)
    scalars = jnp.stack([shift, pos])

    hist_t = jnp.transpose(history, (0, 2, 1))

    out_t = pl.pallas_call(
        _body,
        grid_spec=pltpu.PrefetchScalarGridSpec(
            num_scalar_prefetch=1,
            grid=(),
            in_specs=[
                pl.BlockSpec(memory_space=pltpu.MemorySpace.HBM),
                pl.BlockSpec(memory_space=pltpu.MemorySpace.HBM),
            ],
            out_specs=pl.BlockSpec(memory_space=pltpu.MemorySpace.HBM),
            scratch_shapes=[
                pltpu.VMEM((_D, B, N), history.dtype),
                pltpu.VMEM((_D, B, N), history.dtype),
                pltpu.SemaphoreType.DMA((_D,)),
                pltpu.SemaphoreType.DMA((_D,)),
            ],
        ),
        out_shape=jax.ShapeDtypeStruct((B, H, N), history.dtype),
    )(scalars, hist_t, activations)
    return jnp.transpose(out_t, (0, 2, 1))
